# baseline (speedup 1.0000x reference)
import functools

import jax
import jax.numpy as jnp
from jax import lax
from jax.experimental import pallas as pl
from jax.experimental.pallas import tpu as pltpu
from jax.experimental.pallas import tpu_sc as plsc

_B = 1024
_D = 1024
_N_POP = 256
_T = 256
_TAU = 10.0
_SCALE = _T * _TAU / (_TAU + 1.0)

_NC = 2
_NS = 16
_NW = _NC * _NS
_RPW = _B // _NW
_ROW_WORDS = _T * _N_POP


def _offsets_body(x_ref, w_ref, b_ref, off_ref):
    z = jnp.dot(x_ref[...], w_ref[...], preferred_element_type=jnp.float32)
    intensity = jax.nn.sigmoid(z + b_ref[...])
    st = jnp.clip(((1.0 - intensity) * _SCALE).astype(jnp.int32), 0, _T - 1)
    n_iota = lax.broadcasted_iota(jnp.int32, (_B, _N_POP), 1)
    off_ref[...] = st * _N_POP + n_iota


def _sc_scatter_body(off_ref, out_ref, blk, idx_v):
    c = lax.axis_index("c")
    s = lax.axis_index("s")
    base = (s * _NC + c) * _RPW

    zeros16 = jnp.zeros((16,), jnp.float32)
    ones16 = jnp.ones((16,), jnp.float32)

    def memset_body(i, carry):
        blk[pl.ds(i * 16, 16)] = zeros16
        return carry

    lax.fori_loop(0, _ROW_WORDS // 16, memset_body, 0)

    def row_body(r, carry):
        row = base + r
        pltpu.sync_copy(off_ref.at[row], idx_v)
        for j in range(_N_POP // 16):
            ix = idx_v[pl.ds(j * 16, 16)]
            plsc.store_scatter(blk, [ix], ones16)
        pltpu.sync_copy(blk, out_ref.at[row])
        for j in range(_N_POP // 16):
            ix = idx_v[pl.ds(j * 16, 16)]
            plsc.store_scatter(blk, [ix], zeros16)
        return carry

    lax.fori_loop(0, _RPW, row_body, 0)


@functools.partial(jax.jit)
def kernel(x, W, b):
    wt = W.T
    b2 = b.reshape(1, _N_POP)
    offs = pl.pallas_call(
        _offsets_body,
        grid=(1,),
        in_specs=[
            pl.BlockSpec((_B, _D), lambda i: (0, 0)),
            pl.BlockSpec((_D, _N_POP), lambda i: (0, 0)),
            pl.BlockSpec((1, _N_POP), lambda i: (0, 0)),
        ],
        out_specs=pl.BlockSpec((_B, _N_POP), lambda i: (0, 0)),
        out_shape=jax.ShapeDtypeStruct((_B, _N_POP), jnp.int32),
    )(x, wt, b2)

    sc = pl.kernel(
        _sc_scatter_body,
        out_type=jax.ShapeDtypeStruct((_B, _ROW_WORDS), jnp.float32),
        mesh=plsc.VectorSubcoreMesh(core_axis_name="c", subcore_axis_name="s"),
        scratch_types=[
            pltpu.VMEM((_ROW_WORDS,), jnp.float32),
            pltpu.VMEM((_N_POP,), jnp.int32),
        ],
        compiler_params=pltpu.CompilerParams(needs_layout_passes=False),
    )
    flat = sc(offs)
    return flat.reshape(_B, _T, _N_POP)
